# Initial kernel scaffold; baseline (speedup 1.0000x reference)
import jax
import jax.numpy as jnp
from jax.experimental import pallas as pl


def kernel(indices, values_weight):
    raise NotImplementedError("write your pallas kernel here")


# speedup vs baseline: 1.5582x; 1.5582x over previous
import functools

import jax
import jax.numpy as jnp
from jax import lax
from jax.experimental import pallas as pl
from jax.experimental.pallas import tpu as pltpu
from jax.experimental.pallas import tpu_sc as plsc

D = 32
GRP = 128
GPM = 8
NW = 32


def _sc_gather(idx3, table):
    _, NG, _ = idx3.shape
    n_total = NW * NG * GRP
    n_macro = NG // GPM
    per_w = NG * GRP
    mesh = plsc.VectorSubcoreMesh(core_axis_name="c", subcore_axis_name="s")

    @functools.partial(
        pl.kernel,
        out_type=jax.ShapeDtypeStruct((n_total, D), jnp.float32),
        mesh=mesh,
        compiler_params=pltpu.CompilerParams(use_tc_tiling_on_sc=False),
        scratch_types=[
            pltpu.VMEM((NG, GRP), jnp.int32),
            pltpu.VMEM((GPM * GRP, D), jnp.float32),
            pltpu.SemaphoreType.DMA,
        ],
    )
    def k(idx_hbm, table_hbm, out_hbm, idx_v, rows_v, sem):
        wid = lax.axis_index("s") * 2 + lax.axis_index("c")
        base = wid * per_w
        pltpu.sync_copy(idx_hbm.at[wid], idx_v)

        def body(m, carry):
            copies = []
            for g in range(GPM):
                copies.append(pltpu.async_copy(
                    table_hbm.at[idx_v.at[m * GPM + g]],
                    rows_v.at[pl.ds(g * GRP, GRP)],
                    sem,
                ))
            for cp in copies:
                cp.wait()
            pltpu.sync_copy(
                rows_v,
                out_hbm.at[pl.ds(base + m * (GPM * GRP), GPM * GRP)],
            )
            return carry

        lax.fori_loop(0, n_macro, body, 0)

    return k(idx3, table)


def kernel(indices, values_weight):
    B, F = indices.shape
    n = B * F
    idx3 = indices.astype(jnp.int32).reshape(NW, n // (NW * GRP), GRP)
    out = _sc_gather(idx3, values_weight)
    return out.reshape(B, F, D)


# speedup vs baseline: 1.5780x; 1.0127x over previous
import functools

import jax
import jax.numpy as jnp
from jax import lax
from jax.experimental import pallas as pl
from jax.experimental.pallas import tpu as pltpu
from jax.experimental.pallas import tpu_sc as plsc

D = 32
GRP = 128
NW = 32


def _sc_gather(idxT, table):
    F, B = idxT.shape
    BW = B // NW
    NG = BW // GRP
    mesh = plsc.VectorSubcoreMesh(core_axis_name="c", subcore_axis_name="s")

    @functools.partial(
        pl.kernel,
        out_type=jax.ShapeDtypeStruct((B, F, D), jnp.float32),
        mesh=mesh,
        compiler_params=pltpu.CompilerParams(use_tc_tiling_on_sc=False),
        scratch_types=[
            pltpu.VMEM((F, BW), jnp.int32),
            pltpu.VMEM((2, BW, D), jnp.float32),
            pltpu.SemaphoreType.DMA,
            pltpu.SemaphoreType.DMA,
        ],
    )
    def k(idxT_hbm, table_hbm, out_hbm, idx_v, rows_v, sem0, sem1):
        sems = (sem0, sem1)
        wid = lax.axis_index("s") * 2 + lax.axis_index("c")
        b0 = wid * BW
        pltpu.sync_copy(idxT_hbm.at[:, pl.ds(b0, BW)], idx_v)

        def fire(f, buf):
            for g in range(NG):
                pltpu.async_copy(
                    table_hbm.at[idx_v.at[f, pl.ds(g * GRP, GRP)]],
                    rows_v.at[buf, pl.ds(g * GRP, GRP)],
                    sems[buf],
                )

        def drain(buf):
            for g in range(NG):
                pltpu.make_async_copy(
                    table_hbm.at[pl.ds(0, GRP)],
                    rows_v.at[buf, pl.ds(g * GRP, GRP)],
                    sems[buf],
                ).wait()

        def store(f, buf):
            pltpu.sync_copy(
                rows_v.at[buf],
                out_hbm.at[pl.ds(b0, BW), f, :],
            )

        fire(0, 0)

        def body(i, carry):
            f0 = 2 * i
            fire(f0 + 1, 1)
            drain(0)
            store(f0, 0)
            fire(f0 + 2, 0)
            drain(1)
            store(f0 + 1, 1)
            return carry

        lax.fori_loop(0, (F - 2) // 2, body, 0)
        fire(F - 1, 1)
        drain(0)
        store(F - 2, 0)
        drain(1)
        store(F - 1, 1)

    return k(idxT, table)


def kernel(indices, values_weight):
    idxT = indices.astype(jnp.int32).T
    return _sc_gather(idxT, values_weight)
